# baseline (speedup 1.0000x reference)
import jax
import jax.numpy as jnp
from jax.experimental import pallas as pl

B, N, S, D = 8, 512, 256, 64
NB = 128
B1 = 6


def _tc_body(tt_ref, st_ref, out_ref):
    tt = tt_ref[...]
    st = st_ref[...]
    out_ref[0] = st[:, :, None] + tt[:, None, :]


def _tc_call(tt, st, nbatch):
    return pl.pallas_call(
        _tc_body,
        grid=(nbatch, N // NB),
        in_specs=[
            pl.BlockSpec((D, S), lambda b, j: (0, 0)),
            pl.BlockSpec((D, NB), lambda b, j: (0, j)),
        ],
        out_specs=pl.BlockSpec((1, D, NB, S), lambda b, j: (b, 0, j, 0)),
        out_shape=jax.ShapeDtypeStruct((nbatch, D, N, S), jnp.float32),
    )(tt, st)


def kernel(input_ids, time_table, space_table):
    del input_ids
    tt = time_table[:S].T
    st = space_table.T
    a = _tc_call(tt, st, B1)
    b = _tc_call(tt, st, B - B1)
    return jnp.concatenate([a, b], axis=0)


# speedup vs baseline: 3.0122x; 3.0122x over previous
import jax
import jax.numpy as jnp
from jax.experimental import pallas as pl

B, N, S, D = 8, 512, 256, 64
DB = 16


def _tc_body(tt_ref, st_ref, out_ref):
    tt = tt_ref[...]
    st = st_ref[...]
    out_ref[0] = st[:, :, None] + tt[:, None, :]


def kernel(input_ids, time_table, space_table):
    del input_ids
    tt = time_table[:S].T
    st = space_table.T
    return pl.pallas_call(
        _tc_body,
        grid=(B, D // DB),
        in_specs=[
            pl.BlockSpec((DB, S), lambda b, j: (j, 0)),
            pl.BlockSpec((DB, N), lambda b, j: (j, 0)),
        ],
        out_specs=pl.BlockSpec((1, DB, N, S), lambda b, j: (b, j, 0, 0)),
        out_shape=jax.ShapeDtypeStruct((B, D, N, S), jnp.float32),
    )(tt, st)
